# baseline (speedup 1.0000x reference)
import functools

import jax
import jax.numpy as jnp
import numpy as np
from jax import lax
from jax.experimental import pallas as pl
from jax.experimental.pallas import tpu as pltpu
from jax.experimental.pallas import tpu_sc as plsc

_B, _T, _F = 8, 2048, 1024
_START = 102
_N = _B * _T
_NW = 32
_FRAMES_PER_W = _N // _NW
_C = 16
_CHUNKS = _FRAMES_PER_W // _C
_CW = _C * _F
_L = 16
_VECS = _CW // _L
_TRI = 341
_CIW = (_TRI + 1) * _L

_idx_cache = None


def _rotl(x, d):
    return ((x << np.uint32(d)) | (x >> np.uint32(32 - d))).astype(np.uint32)


def _threefry2x32(k1, k2, x0, x1):
    rotations = ((13, 15, 26, 6), (17, 29, 16, 24))
    ks = [np.uint32(k1), np.uint32(k2), np.uint32(k1 ^ k2 ^ 0x1BD11BDA)]
    x0 = (x0 + ks[0]).astype(np.uint32)
    x1 = (x1 + ks[1]).astype(np.uint32)
    for i in range(5):
        for r in rotations[i % 2]:
            x0 = (x0 + x1).astype(np.uint32)
            x1 = x0 ^ _rotl(x1, r)
        x0 = (x0 + ks[(i + 1) % 3]).astype(np.uint32)
        x1 = (x1 + ks[(i + 2) % 3] + np.uint32(i + 1)).astype(np.uint32)
    return x0, x1


def _np_uniform_key0(shape):
    size = int(np.prod(shape))
    cnt = np.arange(size, dtype=np.uint64)
    hi = (cnt >> np.uint64(32)).astype(np.uint32)
    lo = cnt.astype(np.uint32)
    x0, x1 = _threefry2x32(0, 0, hi, lo)
    bits = x0 ^ x1
    f = ((bits >> np.uint32(9)) | np.uint32(0x3F800000)).view(np.float32)
    return (f - np.float32(1.0)).reshape(shape)


def _local_indices():
    global _idx_cache
    if _idx_cache is None:
        r_np = _np_uniform_key0((_B, _T, _F - _START))
        hf = np.argsort(r_np, axis=-1, kind="stable") + _START
        perm = np.broadcast_to(
            np.arange(_F, dtype=np.int64), (_B, _T, _F)).copy()
        perm[..., _START:] = hf
        perm = perm.reshape(_N, _F)
        groups = perm.reshape(_N // _C, _C * _F // _L, _L).astype(np.int64)
        tri = groups[:, :_TRI * 3, :].reshape(-1, _TRI, 3, _L)
        words3 = tri[:, :, 0, :] | (tri[:, :, 1, :] << 10) \
            | (tri[:, :, 2, :] << 20)
        flat = np.concatenate(
            [words3.reshape(_N // _C, _TRI * _L), groups[:, -1, :]], axis=1)
        assert flat.shape == (_N // _C, _CIW)
        _idx_cache = np.ascontiguousarray(
            flat.reshape(-1).astype(np.int32))
    return _idx_cache


def _sc_gather(x2, idx2):
    mesh = plsc.VectorSubcoreMesh(core_axis_name="c", subcore_axis_name="s")

    @functools.partial(
        pl.kernel,
        mesh=mesh,
        out_type=jax.ShapeDtypeStruct((_N, _F), jnp.float32),
        compiler_params=pltpu.CompilerParams(needs_layout_passes=False),
        scratch_types=[
            pltpu.VMEM((_C, _F), jnp.float32),
            pltpu.VMEM((_C, _F), jnp.float32),
            pltpu.VMEM((_C, _F), jnp.float32),
            pltpu.VMEM((_CIW,), jnp.int32),
            pltpu.VMEM((_CIW,), jnp.int32),
            pltpu.VMEM((_CIW,), jnp.int32),
            pltpu.VMEM((_C, _F), jnp.float32),
            pltpu.VMEM((_C, _F), jnp.float32),
        ] + [pltpu.SemaphoreType.DMA] * 8,
    )
    def k(x_hbm, idx_hbm, out_hbm,
          xv0, xv1, xv2, iv0, iv1, iv2, ov0, ov1, *sems):
        xv = (xv0, xv1, xv2)
        iv = (iv0, iv1, iv2)
        ov = (ov0, ov1)
        sx = sems[0:3]
        si = sems[3:6]
        so = sems[6:8]
        wid = lax.axis_index("s") * 2 + lax.axis_index("c")
        wrow = wid * _FRAMES_PER_W

        def start_load(ci, s):
            r0 = wrow + ci * _C
            dx = pltpu.async_copy(x_hbm.at[pl.ds(r0, _C), :], xv[s], sx[s])
            ib = (r0 // _C) * _CIW
            di = pltpu.async_copy(
                idx_hbm.at[pl.ds(ib, _CIW)], iv[s], si[s])
            return dx, di

        loads = [None, None, None]
        stores = [None, None]
        loads[0] = start_load(0, 0)
        loads[1] = start_load(1, 1)
        for ci in range(_CHUNKS):
            s = ci % 3
            so_s = ci % 2
            if ci + 2 < _CHUNKS:
                loads[(ci + 2) % 3] = start_load(ci + 2, (ci + 2) % 3)
            dx, di = loads[s]
            dx.wait()
            di.wait()
            if stores[so_s] is not None:
                stores[so_s].wait()
            xvs, ivs, ovs = xv[s], iv[s], ov[so_s]

            @plsc.parallel_loop(0, _TRI, unroll=4)
            def tri_body(t):
                w3 = ivs[pl.ds(t * _L, _L)]
                cols = (w3 & 1023, (w3 >> 10) & 1023, w3 >> 20)
                g0 = t * 3
                for i in range(3):
                    g = g0 + i
                    r = g >> 6
                    f0 = (g & 63) * _L
                    rows = jnp.full((_L,), r, dtype=jnp.int32)
                    ovs[r, pl.ds(f0, _L)] = plsc.load_gather(
                        xvs, [rows, cols[i]])

            wt = ivs[pl.ds(_TRI * _L, _L)]
            last = jnp.full((_L,), _C - 1, dtype=jnp.int32)
            ovs[_C - 1, pl.ds(63 * _L, _L)] = plsc.load_gather(
                xvs, [last, wt])

            stores[so_s] = pltpu.async_copy(
                ov[so_s], out_hbm.at[pl.ds(wrow + ci * _C, _C), :], so[so_s])
        stores[0].wait()
        stores[1].wait()

    return k(x2, idx2)


def kernel(x):
    idx = jnp.asarray(_local_indices())
    out = _sc_gather(x.reshape(_N, _F), idx)
    return out.reshape(_B, _T, _F)


# speedup vs baseline: 1.0793x; 1.0793x over previous
import functools

import jax
import jax.numpy as jnp
import numpy as np
from jax import lax
from jax.experimental import pallas as pl
from jax.experimental.pallas import tpu as pltpu
from jax.experimental.pallas import tpu_sc as plsc

_B, _T, _F = 8, 2048, 1024
_START = 102
_N = _B * _T
_NW = 32
_FRAMES_PER_W = _N // _NW
_C = 16
_CHUNKS = _FRAMES_PER_W // _C
_CW = _C * _F
_L = 16
_VECS = _CW // _L

_idx_cache = None


def _rotl(x, d):
    return ((x << np.uint32(d)) | (x >> np.uint32(32 - d))).astype(np.uint32)


def _threefry2x32(k1, k2, x0, x1):
    rotations = ((13, 15, 26, 6), (17, 29, 16, 24))
    ks = [np.uint32(k1), np.uint32(k2), np.uint32(k1 ^ k2 ^ 0x1BD11BDA)]
    x0 = (x0 + ks[0]).astype(np.uint32)
    x1 = (x1 + ks[1]).astype(np.uint32)
    for i in range(5):
        for r in rotations[i % 2]:
            x0 = (x0 + x1).astype(np.uint32)
            x1 = x0 ^ _rotl(x1, r)
        x0 = (x0 + ks[(i + 1) % 3]).astype(np.uint32)
        x1 = (x1 + ks[(i + 2) % 3] + np.uint32(i + 1)).astype(np.uint32)
    return x0, x1


def _np_uniform_key0(shape):
    size = int(np.prod(shape))
    cnt = np.arange(size, dtype=np.uint64)
    hi = (cnt >> np.uint64(32)).astype(np.uint32)
    lo = cnt.astype(np.uint32)
    x0, x1 = _threefry2x32(0, 0, hi, lo)
    bits = x0 ^ x1
    f = ((bits >> np.uint32(9)) | np.uint32(0x3F800000)).view(np.float32)
    return (f - np.float32(1.0)).reshape(shape)


def _local_indices():
    global _idx_cache
    if _idx_cache is None:
        r_np = _np_uniform_key0((_B, _T, _F - _START))
        hf = np.argsort(r_np, axis=-1, kind="stable") + _START
        perm = np.broadcast_to(
            np.arange(_F, dtype=np.int64), (_B, _T, _F)).copy()
        perm[..., _START:] = hf
        perm = perm.reshape(_N, _F)
        blocks = perm.reshape(_N, _F // 32, 2, _L).astype(np.int64)
        words = blocks[:, :, 0, :] | (blocks[:, :, 1, :] << 16)
        _idx_cache = np.ascontiguousarray(
            words.reshape(_N * _F // 2).astype(np.int32))
    return _idx_cache


def _sc_gather(x2, idx2):
    mesh = plsc.VectorSubcoreMesh(core_axis_name="c", subcore_axis_name="s")

    @functools.partial(
        pl.kernel,
        mesh=mesh,
        out_type=jax.ShapeDtypeStruct((_N, _F), jnp.float32),
        compiler_params=pltpu.CompilerParams(needs_layout_passes=False),
        scratch_types=[
            pltpu.VMEM((_C, _F), jnp.float32),
            pltpu.VMEM((_C, _F), jnp.float32),
            pltpu.VMEM((_C, _F), jnp.float32),
            pltpu.VMEM((_CW // 2,), jnp.int32),
            pltpu.VMEM((_CW // 2,), jnp.int32),
            pltpu.VMEM((_CW // 2,), jnp.int32),
            pltpu.VMEM((_C, _F), jnp.float32),
            pltpu.VMEM((_C, _F), jnp.float32),
        ] + [pltpu.SemaphoreType.DMA] * 8,
    )
    def k(x_hbm, idx_hbm, out_hbm,
          xv0, xv1, xv2, iv0, iv1, iv2, ov0, ov1, *sems):
        xv = (xv0, xv1, xv2)
        iv = (iv0, iv1, iv2)
        ov = (ov0, ov1)
        sx = sems[0:3]
        si = sems[3:6]
        so = sems[6:8]
        wid = lax.axis_index("s") * 2 + lax.axis_index("c")
        wrow = wid * _FRAMES_PER_W

        def start_load(ci, s):
            r0 = wrow + ci * _C
            dx = pltpu.async_copy(x_hbm.at[pl.ds(r0, _C), :], xv[s], sx[s])
            di = pltpu.async_copy(
                idx_hbm.at[pl.ds(r0 * (_F // 2), _CW // 2)], iv[s], si[s])
            return dx, di

        loads = [None, None, None]
        stores = [None, None]
        loads[0] = start_load(0, 0)
        loads[1] = start_load(1, 1)
        for ci in range(_CHUNKS):
            s = ci % 3
            so_s = ci % 2
            if ci + 2 < _CHUNKS:
                loads[(ci + 2) % 3] = start_load(ci + 2, (ci + 2) % 3)
            dx, di = loads[s]
            dx.wait()
            di.wait()
            if stores[so_s] is not None:
                stores[so_s].wait()
            xvs, ivs, ovs = xv[s], iv[s], ov[so_s]

            @plsc.parallel_loop(0, _VECS // 2, unroll=16)
            def vec_body(j):
                r = j >> 5
                f0 = (j & 31) * (2 * _L)
                w = ivs[pl.ds(j * _L, _L)]
                ca = w & 0xFFFF
                cb = w >> 16
                rows = jnp.full((_L,), r, dtype=jnp.int32)
                ovs[r, pl.ds(f0, _L)] = plsc.load_gather(xvs, [rows, ca])
                ovs[r, pl.ds(f0 + _L, _L)] = plsc.load_gather(
                    xvs, [rows, cb])

            stores[so_s] = pltpu.async_copy(
                ov[so_s], out_hbm.at[pl.ds(wrow + ci * _C, _C), :], so[so_s])
        stores[0].wait()
        stores[1].wait()

    return k(x2, idx2)


def kernel(x):
    idx = jnp.asarray(_local_indices())
    out = _sc_gather(x.reshape(_N, _F), idx)
    return out.reshape(_B, _T, _F)


# speedup vs baseline: 1.1096x; 1.0280x over previous
import functools

import jax
import jax.numpy as jnp
import numpy as np
from jax import lax
from jax.experimental import pallas as pl
from jax.experimental.pallas import tpu as pltpu
from jax.experimental.pallas import tpu_sc as plsc

_B, _T, _F = 8, 2048, 1024
_START = 102
_N = _B * _T
_NW = 32
_FRAMES_PER_W = _N // _NW
_C = 16
_CHUNKS = _FRAMES_PER_W // _C
_CW = _C * _F
_L = 16
_VECS = _CW // _L

_idx_cache = None


def _rotl(x, d):
    return ((x << np.uint32(d)) | (x >> np.uint32(32 - d))).astype(np.uint32)


def _threefry2x32(k1, k2, x0, x1):
    rotations = ((13, 15, 26, 6), (17, 29, 16, 24))
    ks = [np.uint32(k1), np.uint32(k2), np.uint32(k1 ^ k2 ^ 0x1BD11BDA)]
    x0 = (x0 + ks[0]).astype(np.uint32)
    x1 = (x1 + ks[1]).astype(np.uint32)
    for i in range(5):
        for r in rotations[i % 2]:
            x0 = (x0 + x1).astype(np.uint32)
            x1 = x0 ^ _rotl(x1, r)
        x0 = (x0 + ks[(i + 1) % 3]).astype(np.uint32)
        x1 = (x1 + ks[(i + 2) % 3] + np.uint32(i + 1)).astype(np.uint32)
    return x0, x1


def _np_uniform_key0(shape):
    size = int(np.prod(shape))
    cnt = np.arange(size, dtype=np.uint64)
    hi = (cnt >> np.uint64(32)).astype(np.uint32)
    lo = cnt.astype(np.uint32)
    x0, x1 = _threefry2x32(0, 0, hi, lo)
    bits = x0 ^ x1
    f = ((bits >> np.uint32(9)) | np.uint32(0x3F800000)).view(np.float32)
    return (f - np.float32(1.0)).reshape(shape)


def _local_indices():
    global _idx_cache
    if _idx_cache is None:
        r_np = _np_uniform_key0((_B, _T, _F - _START))
        hf = np.argsort(r_np, axis=-1, kind="stable") + _START
        perm = np.broadcast_to(
            np.arange(_F, dtype=np.int64), (_B, _T, _F)).copy()
        perm[..., _START:] = hf
        perm = perm.reshape(_N, _F)
        blocks = perm.reshape(_N, _F // 32, 2, _L).astype(np.int64)
        words = blocks[:, :, 0, :] | (blocks[:, :, 1, :] << 16)
        _idx_cache = np.ascontiguousarray(
            words.reshape(_N * _F // 2).astype(np.int32))
    return _idx_cache


def _sc_gather(x2, idx2):
    mesh = plsc.VectorSubcoreMesh(core_axis_name="c", subcore_axis_name="s")

    @functools.partial(
        pl.kernel,
        mesh=mesh,
        out_type=jax.ShapeDtypeStruct((_N, _F), jnp.float32),
        compiler_params=pltpu.CompilerParams(needs_layout_passes=False),
        scratch_types=[
            pltpu.VMEM((_C, _F), jnp.float32),
            pltpu.VMEM((_C, _F), jnp.float32),
            pltpu.VMEM((_C, _F), jnp.float32),
            pltpu.VMEM((_CW // 2,), jnp.int32),
            pltpu.VMEM((_CW // 2,), jnp.int32),
            pltpu.VMEM((_CW // 2,), jnp.int32),
            pltpu.VMEM((_C, _F), jnp.float32),
            pltpu.VMEM((_C, _F), jnp.float32),
        ] + [pltpu.SemaphoreType.DMA] * 8,
    )
    def k(x_hbm, idx_hbm, out_hbm,
          xv0, xv1, xv2, iv0, iv1, iv2, ov0, ov1, *sems):
        xv = (xv0, xv1, xv2)
        iv = (iv0, iv1, iv2)
        ov = (ov0, ov1)
        sx = sems[0:3]
        si = sems[3:6]
        so = sems[6:8]
        wid = lax.axis_index("s") * 2 + lax.axis_index("c")
        wrow = wid * _FRAMES_PER_W

        def start_load(ci, s):
            r0 = wrow + ci * _C
            dx = pltpu.async_copy(x_hbm.at[pl.ds(r0, _C), :], xv[s], sx[s])
            di = pltpu.async_copy(
                idx_hbm.at[pl.ds(r0 * (_F // 2), _CW // 2)], iv[s], si[s])
            return dx, di

        loads = [None, None, None]
        stores = [None, None]
        loads[0] = start_load(0, 0)
        loads[1] = start_load(1, 1)
        for ci in range(_CHUNKS):
            s = ci % 3
            so_s = ci % 2
            if ci + 2 < _CHUNKS:
                loads[(ci + 2) % 3] = start_load(ci + 2, (ci + 2) % 3)
            dx, di = loads[s]
            dx.wait()
            di.wait()
            if stores[so_s] is not None:
                stores[so_s].wait()
            xvs, ivs, ovs = xv[s], iv[s], ov[so_s]

            @plsc.parallel_loop(0, _VECS // 2, unroll=8)
            def vec_body(j):
                r = j >> 5
                f0 = (j & 31) * (2 * _L)
                w = ivs[pl.ds(j * _L, _L)]
                ca = w & 0xFFFF
                cb = w >> 16
                rows = jnp.full((_L,), r, dtype=jnp.int32)
                ovs[r, pl.ds(f0, _L)] = plsc.load_gather(xvs, [rows, ca])
                ovs[r, pl.ds(f0 + _L, _L)] = plsc.load_gather(
                    xvs, [rows, cb])

            stores[so_s] = pltpu.async_copy(
                ov[so_s], out_hbm.at[pl.ds(wrow + ci * _C, _C), :], so[so_s])
        stores[0].wait()
        stores[1].wait()

    return k(x2, idx2)


def kernel(x):
    idx = jnp.asarray(_local_indices())
    out = _sc_gather(x.reshape(_N, _F), idx)
    return out.reshape(_B, _T, _F)
